# baseline (speedup 1.0000x reference)
import functools

import jax
import jax.numpy as jnp
from jax import lax
from jax.experimental import pallas as pl
from jax.experimental.pallas import tpu as pltpu
from jax.experimental.pallas import tpu_sc as plsc

N = 10000
T = 20
DTS = 32
H = 64
F = 128
GH = 64
GO = 32
E = 320000

NW = 32
NSUB = 16
NP = 10240
RPT = NP // NSUB
CH = 80
K = 125
NBUF = 4

ROWS = 1000
GRID = 10
RMF = 2000
GMF = 5

_f32 = jnp.float32
_bf16 = jnp.bfloat16


def _sc_mesh():
    return plsc.VectorSubcoreMesh(core_axis_name="c", subcore_axis_name="s")


def _sc_degree(edges4, zrow, ones_c):
    @functools.partial(
        pl.kernel,
        out_type=jax.ShapeDtypeStruct((NP,), _f32),
        mesh=_sc_mesh(),
        scratch_types=[
            pltpu.VMEM((2 * K, CH), jnp.int32),
            pltpu.VMEM((CH,), _f32),
            pltpu.VMEM_SHARED((NP,), _f32),
        ],
    )
    def k(edges_hbm, z_hbm, ones_hbm, out_hbm, idx_v, ones_v, acc_sh):
        c = lax.axis_index("c")
        s = lax.axis_index("s")

        @pl.when(c == 0)
        def _():
            pltpu.sync_copy(edges_hbm.at[1, 2 * s], idx_v.at[pl.ds(0, K)])
            pltpu.sync_copy(edges_hbm.at[1, 2 * s + 1], idx_v.at[pl.ds(K, K)])
            pltpu.sync_copy(ones_hbm, ones_v)
            pltpu.sync_copy(z_hbm, acc_sh.at[pl.ds(s * RPT, RPT)])
            plsc.subcore_barrier()

            def body(j, carry):
                pltpu.sync_copy(ones_v, acc_sh.at[idx_v.at[j]], add=True)
                return carry

            lax.fori_loop(0, 2 * K, body, 0)
            plsc.subcore_barrier()
            pltpu.sync_copy(acc_sh.at[pl.ds(s * RPT, RPT)],
                            out_hbm.at[pl.ds(s * RPT, RPT)])

    return k(edges4, zrow, ones_c)


def _sc_scatter(g, edges4, zrows, d):
    @functools.partial(
        pl.kernel,
        out_type=jax.ShapeDtypeStruct((NP, 2 * d), _f32),
        mesh=_sc_mesh(),
        scratch_types=[
            pltpu.VMEM((K, CH), jnp.int32),
            pltpu.VMEM((K, CH), jnp.int32),
            [pltpu.VMEM((CH, d), _f32) for _ in range(NBUF)],
            [pltpu.SemaphoreType.DMA for _ in range(NBUF)],
            pltpu.VMEM_SHARED((NP, d), _f32),
        ],
        compiler_params=pltpu.CompilerParams(use_tc_tiling_on_sc=False),
    )
    def k(g_hbm, edges_hbm, z_hbm, out_hbm, is_v, id_v, rows, sems,
          acc_sh):
        c = lax.axis_index("c")
        s = lax.axis_index("s")
        w = c * NSUB + s
        pltpu.sync_copy(edges_hbm.at[0, w], is_v)
        pltpu.sync_copy(edges_hbm.at[1, w], id_v)
        pltpu.sync_copy(z_hbm, acc_sh.at[pl.ds(s * RPT, RPT)])
        plsc.subcore_barrier()
        for b in range(NBUF):
            pltpu.async_copy(g_hbm.at[is_v.at[b]], rows[b], sems[b])

        def body(i, carry):
            for b in range(NBUF):
                j = NBUF * i + b
                pltpu.make_async_copy(g_hbm.at[is_v.at[j]], rows[b],
                                      sems[b]).wait()
                pltpu.sync_copy(rows[b], acc_sh.at[id_v.at[j]], add=True)

                @pl.when(j + NBUF < K)
                def _():
                    pltpu.async_copy(g_hbm.at[is_v.at[j + NBUF]], rows[b],
                                     sems[b])
            return carry

        lax.fori_loop(0, K // NBUF, body, 0)
        jlast = (K // NBUF) * NBUF
        pltpu.make_async_copy(g_hbm.at[is_v.at[jlast]], rows[0], sems[0]).wait()
        pltpu.sync_copy(rows[0], acc_sh.at[id_v.at[jlast]], add=True)
        plsc.subcore_barrier()
        pltpu.sync_copy(acc_sh.at[pl.ds(s * RPT, RPT)],
                        out_hbm.at[pl.ds(s * RPT, RPT), pl.ds(c * d, d)])

    return k(g, edges4, zrows)


def _row_spec(cols):
    return pl.BlockSpec((ROWS, cols), lambda i: (i, 0))


def _full_spec(r, cols):
    return pl.BlockSpec((r, cols), lambda i: (0, 0))


def _enc_body(ts_ref, st_ref, deg_ref, wcat_ref,
              sw_ref, sb_ref, fwt_ref, fwb_ref, fb_ref, w1_ref,
              g1_ref):
    h = jnp.zeros((ROWS, H), _bf16)
    cst = jnp.zeros((ROWS, H), _f32)
    wcat = wcat_ref[...].astype(_bf16)
    x = ts_ref[...].astype(_bf16)
    ones8 = jnp.ones((ROWS, 8), _bf16)
    for t in range(T):
        xh = jnp.concatenate([x[:, t * DTS:(t + 1) * DTS], h, ones8], -1)
        g = jnp.dot(xh, wcat, preferred_element_type=_f32)
        th = jnp.tanh(g)
        gi = 0.5 * th[:, 0:H] + 0.5
        gf = 0.5 * th[:, H:2 * H] + 0.5
        gg = th[:, 2 * H:3 * H]
        go = 0.5 * th[:, 3 * H:] + 0.5
        cst = gf * cst + gi * gg
        h = (go * jnp.tanh(cst)).astype(_bf16)
    st = jnp.maximum(
        jnp.dot(st_ref[...], sw_ref[...], preferred_element_type=_f32)
        + sb_ref[...], 0.0)
    fused = jnp.maximum(
        jnp.dot(h, fwt_ref[...], preferred_element_type=_f32)
        + jnp.dot(st, fwb_ref[...], preferred_element_type=_f32)
        + fb_ref[...], 0.0)
    hh = jnp.dot(fused, w1_ref[...], preferred_element_type=_f32)
    dinv = lax.rsqrt(deg_ref[...] + 1.0)
    g1_ref[...] = dinv * hh


def _tc_encoder(ts2, static_data, deg, wcat, sw, sb, fwt, fwb, fb, w1):
    return pl.pallas_call(
        _enc_body,
        grid=(GRID,),
        in_specs=[
            _row_spec(T * DTS),
            _row_spec(64),
            _row_spec(1),
            _full_spec(DTS + H + 8, 4 * H),
            _full_spec(64, 32),
            _full_spec(1, 32),
            _full_spec(H, F),
            _full_spec(32, F),
            _full_spec(1, F),
            _full_spec(F, GH),
        ],
        out_specs=_row_spec(GH),
        out_shape=jax.ShapeDtypeStruct((N, GH), _f32),
    )(ts2, static_data, deg, wcat, sw, sb, fwt, fwb, fb, w1)


def _rowmf_spec(cols):
    return pl.BlockSpec((RMF, cols), lambda i: (i, 0))


def _fullmf_spec(r, cols):
    return pl.BlockSpec((r, cols), lambda i: (0, 0))


def _pair_spec(cols):
    return pl.BlockSpec((RMF, 2 * cols), lambda i: (i, 0))


def _mid_body(s_ref, g1_ref, deg_ref, b1_ref, w2_ref, out_ref):
    dinv = lax.rsqrt(deg_ref[...] + 1.0)
    sv = s_ref[...]
    ssum = sv[:, :GH] + sv[:, GH:]
    h1 = jnp.maximum(dinv * (ssum + g1_ref[...]) + b1_ref[...], 0.0)
    out_ref[...] = dinv * jnp.dot(h1, w2_ref[...], preferred_element_type=_f32)


def _tc_mid(s1, g1, deg, b1, w2):
    return pl.pallas_call(
        _mid_body,
        grid=(GMF,),
        in_specs=[
            _pair_spec(GH), _rowmf_spec(GH), _rowmf_spec(1),
            _fullmf_spec(1, GH), _fullmf_spec(GH, GO),
        ],
        out_specs=_rowmf_spec(GO),
        out_shape=jax.ShapeDtypeStruct((N, GO), _f32),
    )(s1, g1, deg, b1, w2)


def _fin_body(s_ref, g2_ref, deg_ref, b2_ref, out_ref):
    dinv = lax.rsqrt(deg_ref[...] + 1.0)
    sv = s_ref[...]
    ssum = sv[:, :GO] + sv[:, GO:]
    out_ref[...] = dinv * (ssum + g2_ref[...]) + b2_ref[...]


def _tc_final(s2, g2, deg, b2):
    return pl.pallas_call(
        _fin_body,
        grid=(GMF,),
        in_specs=[
            _pair_spec(GO), _rowmf_spec(GO), _rowmf_spec(1),
            _fullmf_spec(1, GO),
        ],
        out_specs=_rowmf_spec(GO),
        out_shape=jax.ShapeDtypeStruct((N, GO), _f32),
    )(s2, g2, deg, b2)


def kernel(ts_data, static_data, edge_index, lstm_Wih, lstm_Whh, lstm_bih,
           lstm_bhh, static_W, static_b, fus_W, fus_b, gcn1_W, gcn1_b,
           gcn2_W, gcn2_b):
    edges4 = edge_index.reshape(2, NW, K, CH)

    zrow = jnp.zeros((RPT,), _f32)
    ones_c = jnp.ones((CH,), _f32)
    z64 = jnp.zeros((RPT, GH), _f32)
    z32 = jnp.zeros((RPT, GO), _f32)

    deg = _sc_degree(edges4, zrow, ones_c).reshape(NP, 1)

    gsc = jnp.repeat(jnp.array([0.5, 0.5, 1.0, 0.5], _f32), H)[None]
    wcat = jnp.concatenate(
        [lstm_Wih.T, lstm_Whh.T, (lstm_bih + lstm_bhh)[None],
         jnp.zeros((7, 4 * H), _f32)], axis=0) * gsc

    ts2 = ts_data.reshape(N, T * DTS)
    g1 = _tc_encoder(ts2, static_data, deg, wcat,
                     static_W, static_b[None], fus_W[:H], fus_W[H:],
                     fus_b[None], gcn1_W)

    s1 = _sc_scatter(g1, edges4, z64, GH)
    g2 = _tc_mid(s1, g1, deg, gcn1_b[None], gcn2_W)
    s2 = _sc_scatter(g2, edges4, z32, GO)
    return _tc_final(s2, g2, deg, gcn2_b[None])


# speedup vs baseline: 1.0268x; 1.0268x over previous
import functools

import jax
import jax.numpy as jnp
from jax import lax
from jax.experimental import pallas as pl
from jax.experimental.pallas import tpu as pltpu
from jax.experimental.pallas import tpu_sc as plsc

N = 10000
T = 20
DTS = 32
H = 64
F = 128
GH = 64
GO = 32
E = 320000

NW = 32
NSUB = 16
NP = 10240
RPT = NP // NSUB
CH = 80
K = 125
NBUF = 4

ROWS = 1000
GRID = 10
RMF = 2000
GMF = 5

_f32 = jnp.float32
_bf16 = jnp.bfloat16


def _sc_mesh():
    return plsc.VectorSubcoreMesh(core_axis_name="c", subcore_axis_name="s")


def _sc_degree(edges4, zrow, ones_c):
    @functools.partial(
        pl.kernel,
        out_type=jax.ShapeDtypeStruct((NP,), _f32),
        mesh=_sc_mesh(),
        scratch_types=[
            pltpu.VMEM((2 * K, CH), jnp.int32),
            pltpu.VMEM((CH,), _f32),
            pltpu.VMEM_SHARED((NP,), _f32),
        ],
        compiler_params=pltpu.CompilerParams(use_tc_tiling_on_sc=False),
    )
    def k(edges_hbm, z_hbm, ones_hbm, out_hbm, idx_v, ones_v, acc_sh):
        c = lax.axis_index("c")
        s = lax.axis_index("s")

        @pl.when(c == 0)
        def _():
            pltpu.sync_copy(edges_hbm.at[1, 2 * s], idx_v.at[pl.ds(0, K)])
            pltpu.sync_copy(edges_hbm.at[1, 2 * s + 1], idx_v.at[pl.ds(K, K)])
            pltpu.sync_copy(ones_hbm, ones_v)
            pltpu.sync_copy(z_hbm, acc_sh.at[pl.ds(s * RPT, RPT)])
            plsc.subcore_barrier()

            def body(j, carry):
                pltpu.sync_copy(ones_v, acc_sh.at[idx_v.at[j]], add=True)
                return carry

            lax.fori_loop(0, 2 * K, body, 0)
            plsc.subcore_barrier()
            pltpu.sync_copy(acc_sh.at[pl.ds(s * RPT, RPT)],
                            out_hbm.at[pl.ds(s * RPT, RPT)])

    return k(edges4, zrow, ones_c)


def _sc_scatter(g, edges4, zrows, d):
    @functools.partial(
        pl.kernel,
        out_type=jax.ShapeDtypeStruct((NP, 2 * d), _f32),
        mesh=_sc_mesh(),
        scratch_types=[
            pltpu.VMEM((K, CH), jnp.int32),
            pltpu.VMEM((K, CH), jnp.int32),
            [pltpu.VMEM((CH, d), _f32) for _ in range(NBUF)],
            [pltpu.SemaphoreType.DMA for _ in range(NBUF)],
            pltpu.VMEM_SHARED((NP, d), _f32),
        ],
        compiler_params=pltpu.CompilerParams(use_tc_tiling_on_sc=False),
    )
    def k(g_hbm, edges_hbm, z_hbm, out_hbm, is_v, id_v, rows, sems,
          acc_sh):
        c = lax.axis_index("c")
        s = lax.axis_index("s")
        w = c * NSUB + s
        pltpu.sync_copy(edges_hbm.at[0, w], is_v)
        pltpu.sync_copy(edges_hbm.at[1, w], id_v)
        pltpu.sync_copy(z_hbm, acc_sh.at[pl.ds(s * RPT, RPT)])
        plsc.subcore_barrier()
        for b in range(NBUF):
            pltpu.async_copy(g_hbm.at[is_v.at[b]], rows[b], sems[b])

        def body(i, carry):
            for b in range(NBUF):
                j = NBUF * i + b
                pltpu.make_async_copy(g_hbm.at[is_v.at[j]], rows[b],
                                      sems[b]).wait()
                pltpu.sync_copy(rows[b], acc_sh.at[id_v.at[j]], add=True)

                @pl.when(j + NBUF < K)
                def _():
                    pltpu.async_copy(g_hbm.at[is_v.at[j + NBUF]], rows[b],
                                     sems[b])
            return carry

        lax.fori_loop(0, K // NBUF, body, 0)
        jlast = (K // NBUF) * NBUF
        pltpu.make_async_copy(g_hbm.at[is_v.at[jlast]], rows[0], sems[0]).wait()
        pltpu.sync_copy(rows[0], acc_sh.at[id_v.at[jlast]], add=True)
        plsc.subcore_barrier()
        pltpu.sync_copy(acc_sh.at[pl.ds(s * RPT, RPT)],
                        out_hbm.at[pl.ds(s * RPT, RPT), pl.ds(c * d, d)])

    return k(g, edges4, zrows)


def _row_spec(cols):
    return pl.BlockSpec((ROWS, cols), lambda i: (i, 0))


def _full_spec(r, cols):
    return pl.BlockSpec((r, cols), lambda i: (0, 0))


def _enc_body(ts_ref, st_ref, deg_ref, wcat_ref,
              sw_ref, sb_ref, fwt_ref, fwb_ref, fb_ref, w1_ref,
              g1_ref):
    h = jnp.zeros((ROWS, H), _bf16)
    cst = jnp.zeros((ROWS, H), _f32)
    wcat = wcat_ref[...].astype(_bf16)
    x = ts_ref[...].astype(_bf16)
    ones8 = jnp.ones((ROWS, 8), _bf16)
    for t in range(T):
        xh = jnp.concatenate([x[:, t * DTS:(t + 1) * DTS], h, ones8], -1)
        g = jnp.dot(xh, wcat, preferred_element_type=_f32)
        th = jnp.tanh(g)
        gi = 0.5 * th[:, 0:H] + 0.5
        gf = 0.5 * th[:, H:2 * H] + 0.5
        gg = th[:, 2 * H:3 * H]
        go = 0.5 * th[:, 3 * H:] + 0.5
        cst = gf * cst + gi * gg
        h = (go * jnp.tanh(cst)).astype(_bf16)
    st = jnp.maximum(
        jnp.dot(st_ref[...], sw_ref[...], preferred_element_type=_f32)
        + sb_ref[...], 0.0)
    fused = jnp.maximum(
        jnp.dot(h, fwt_ref[...], preferred_element_type=_f32)
        + jnp.dot(st, fwb_ref[...], preferred_element_type=_f32)
        + fb_ref[...], 0.0)
    hh = jnp.dot(fused, w1_ref[...], preferred_element_type=_f32)
    dinv = lax.rsqrt(deg_ref[...] + 1.0)
    g1_ref[...] = dinv * hh


def _tc_encoder(ts2, static_data, deg, wcat, sw, sb, fwt, fwb, fb, w1):
    return pl.pallas_call(
        _enc_body,
        grid=(GRID,),
        in_specs=[
            _row_spec(T * DTS),
            _row_spec(64),
            _row_spec(1),
            _full_spec(DTS + H + 8, 4 * H),
            _full_spec(64, 32),
            _full_spec(1, 32),
            _full_spec(H, F),
            _full_spec(32, F),
            _full_spec(1, F),
            _full_spec(F, GH),
        ],
        out_specs=_row_spec(GH),
        out_shape=jax.ShapeDtypeStruct((N, GH), _f32),
    )(ts2, static_data, deg, wcat, sw, sb, fwt, fwb, fb, w1)


def _rowmf_spec(cols):
    return pl.BlockSpec((RMF, cols), lambda i: (i, 0))


def _fullmf_spec(r, cols):
    return pl.BlockSpec((r, cols), lambda i: (0, 0))


def _pair_spec(cols):
    return pl.BlockSpec((RMF, 2 * cols), lambda i: (i, 0))


def _mid_body(s_ref, g1_ref, deg_ref, b1_ref, w2_ref, out_ref):
    dinv = lax.rsqrt(deg_ref[...] + 1.0)
    sv = s_ref[...]
    ssum = sv[:, :GH] + sv[:, GH:]
    h1 = jnp.maximum(dinv * (ssum + g1_ref[...]) + b1_ref[...], 0.0)
    out_ref[...] = dinv * jnp.dot(h1, w2_ref[...], preferred_element_type=_f32)


def _tc_mid(s1, g1, deg, b1, w2):
    return pl.pallas_call(
        _mid_body,
        grid=(GMF,),
        in_specs=[
            _pair_spec(GH), _rowmf_spec(GH), _rowmf_spec(1),
            _fullmf_spec(1, GH), _fullmf_spec(GH, GO),
        ],
        out_specs=_rowmf_spec(GO),
        out_shape=jax.ShapeDtypeStruct((N, GO), _f32),
    )(s1, g1, deg, b1, w2)


def _fin_body(s_ref, g2_ref, deg_ref, b2_ref, out_ref):
    dinv = lax.rsqrt(deg_ref[...] + 1.0)
    sv = s_ref[...]
    ssum = sv[:, :GO] + sv[:, GO:]
    out_ref[...] = dinv * (ssum + g2_ref[...]) + b2_ref[...]


def _tc_final(s2, g2, deg, b2):
    return pl.pallas_call(
        _fin_body,
        grid=(GMF,),
        in_specs=[
            _pair_spec(GO), _rowmf_spec(GO), _rowmf_spec(1),
            _fullmf_spec(1, GO),
        ],
        out_specs=_rowmf_spec(GO),
        out_shape=jax.ShapeDtypeStruct((N, GO), _f32),
    )(s2, g2, deg, b2)


def kernel(ts_data, static_data, edge_index, lstm_Wih, lstm_Whh, lstm_bih,
           lstm_bhh, static_W, static_b, fus_W, fus_b, gcn1_W, gcn1_b,
           gcn2_W, gcn2_b):
    edges4 = edge_index.reshape(2, NW, K, CH)

    zrow = jnp.zeros((RPT,), _f32)
    ones_c = jnp.ones((CH,), _f32)
    z64 = jnp.zeros((RPT, GH), _f32)
    z32 = jnp.zeros((RPT, GO), _f32)

    deg = _sc_degree(edges4, zrow, ones_c).reshape(NP, 1)

    gsc = jnp.repeat(jnp.array([0.5, 0.5, 1.0, 0.5], _f32), H)[None]
    wcat = jnp.concatenate(
        [lstm_Wih.T, lstm_Whh.T, (lstm_bih + lstm_bhh)[None],
         jnp.zeros((7, 4 * H), _f32)], axis=0) * gsc

    ts2 = ts_data.reshape(N, T * DTS)
    g1 = _tc_encoder(ts2, static_data, deg, wcat,
                     static_W, static_b[None], fus_W[:H], fus_W[H:],
                     fus_b[None], gcn1_W)

    s1 = _sc_scatter(g1, edges4, z64, GH)
    g2 = _tc_mid(s1, g1, deg, gcn1_b[None], gcn2_W)
    s2 = _sc_scatter(g2, edges4, z32, GO)
    return _tc_final(s2, g2, deg, gcn2_b[None])
